# baseline (speedup 1.0000x reference)
import functools

import jax
import jax.numpy as jnp
from jax import lax
from jax.experimental import pallas as pl
from jax.experimental.pallas import tpu as pltpu
from jax.experimental.pallas import tpu_sc as plsc

N = 10000
K = 16
C = 64
NPAD = 10240
BQ = 1024
CW = 256
NC = NPAD // CW
BIGI = NPAD

NW = 32
RPW = NPAD // NW
GB = 8
NG = RPW // GB


NEX = 5
FBIG = float(BIGI)
FBIG2 = float(1 << 24)


def _knn_body(qx_ref, qy_ref, qz_ref, px_ref, py_ref, pz_ref, idx_ref):
    qx = qx_ref[...]
    qy = qy_ref[...]
    qz = qz_ref[...]
    lane = lax.broadcasted_iota(
        jnp.int32, (BQ, CW), 1).astype(jnp.float32)
    lane16 = lax.broadcasted_iota(
        jnp.int32, (BQ, K), 1).astype(jnp.float32)
    lanec = lax.broadcasted_iota(
        jnp.int32, (BQ, NC), 1).astype(jnp.float32)
    inf = jnp.inf

    def dist_chunk(jj):
        dx = qx - px_ref[jj]
        dy = qy - py_ref[jj]
        dz = qz - pz_ref[jj]
        return (dx * dx + dy * dy) + dz * dz

    def extract(j, cc):
        vs, is_, vl, plast = cc
        for u in range(4):
            jj = 4 * j + u
            jf = jj.astype(jnp.float32)
            blk = dist_chunk(jj)
            excl = jnp.zeros((BQ, CW), jnp.bool_)
            vs2, is2 = [], []
            for _ in range(NEX):
                v = jnp.min(jnp.where(excl, inf, blk),
                            axis=1, keepdims=True)
                p = jnp.min(jnp.where((blk == v) & ~excl, lane, FBIG),
                            axis=1, keepdims=True)
                excl = excl | (lane == p)
                vs2.append(v)
                is2.append(p + jf * CW)
            sel = lanec == jf
            vs = [jnp.where(sel, v2, a) for a, v2 in zip(vs, vs2)]
            is_ = [jnp.where(sel, i2, a) for a, i2 in zip(is_, is2)]
            vl = jnp.where(sel, v, vl)
            plast = jnp.where(sel, p, plast)
        return vs, is_, vl, plast

    vs, is_, vl, plast = lax.fori_loop(
        0, NC // 4, extract,
        ([jnp.full((BQ, NC), inf, jnp.float32) for _ in range(NEX)],
         [jnp.full((BQ, NC), FBIG2, jnp.float32) for _ in range(NEX)],
         jnp.full((BQ, NC), inf, jnp.float32),
         jnp.full((BQ, NC), FBIG, jnp.float32)))

    def top16(comb_v, comb_i, tv, ti):
        cv = jnp.full((BQ, 1), -inf, jnp.float32)
        ci = jnp.full((BQ, 1), -1.0, jnp.float32)
        for k in range(K):
            candm = (comb_v > cv) | ((comb_v == cv) & (comb_i > ci))
            m = jnp.min(jnp.where(candm, comb_v, inf),
                        axis=1, keepdims=True)
            ai = jnp.min(jnp.where((comb_v == m) & candm, comb_i, FBIG2),
                         axis=1, keepdims=True)
            tv = jnp.where(lane16 == k, m, tv)
            ti = jnp.where(lane16 == k, ai, ti)
            cv, ci = m, ai
        return tv, ti, cv

    tv, ti, t16 = top16(
        jnp.concatenate(vs, axis=1), jnp.concatenate(is_, axis=1),
        jnp.full((BQ, K), inf, jnp.float32),
        jnp.full((BQ, K), FBIG2, jnp.float32))

    def sweep_cond(carry):
        return carry[0] > 0.5

    def sweep_body(carry):
        _, tv, ti, t16, vl, plast = carry

        def chunk_body(j, cc):
            cvs, cis, vl, plast = cc
            jf = j.astype(jnp.float32)
            blk = dist_chunk(j)
            sel = lanec == jf
            vj = jnp.max(jnp.where(sel, vl, -inf), axis=1, keepdims=True)
            pj = jnp.max(jnp.where(sel, plast, -inf), axis=1, keepdims=True)
            live = vj <= t16
            cand = ((blk > vj) | ((blk == vj) & (lane > pj))) & live
            v = jnp.min(jnp.where(cand, blk, inf), axis=1, keepdims=True)
            p = jnp.min(jnp.where((blk == v) & cand, lane, FBIG),
                        axis=1, keepdims=True)
            gid = jnp.where(v < inf, p + jf * CW, FBIG2)
            cvs = jnp.where(sel, v, cvs)
            cis = jnp.where(sel, gid, cis)
            vl = jnp.where(sel, jnp.where(live, v, vj), vl)
            plast = jnp.where(sel, jnp.where(live, p, pj), plast)
            return cvs, cis, vl, plast

        cvs, cis, vl, plast = lax.fori_loop(
            0, NC, chunk_body,
            (jnp.full((BQ, NC), inf, jnp.float32),
             jnp.full((BQ, NC), FBIG2, jnp.float32), vl, plast))
        tv, ti, t16 = top16(jnp.concatenate([tv, cvs], axis=1),
                            jnp.concatenate([ti, cis], axis=1), tv, ti)
        rem = jnp.sum((vl <= t16).astype(jnp.float32))
        return rem, tv, ti, t16, vl, plast

    rem0 = jnp.sum((vl <= t16).astype(jnp.float32))
    _, _, ti, _, _, _ = lax.while_loop(
        sweep_cond, sweep_body, (rem0, tv, ti, t16, vl, plast))
    idx_ref[...] = ti.astype(jnp.int32)


def _knn(qx, qy, qz, px, py, pz):
    return pl.pallas_call(
        _knn_body,
        grid=(NPAD // BQ,),
        in_specs=[
            pl.BlockSpec((BQ, 1), lambda i: (i, 0)),
            pl.BlockSpec((BQ, 1), lambda i: (i, 0)),
            pl.BlockSpec((BQ, 1), lambda i: (i, 0)),
            pl.BlockSpec((NC, 1, CW), lambda i: (0, 0, 0)),
            pl.BlockSpec((NC, 1, CW), lambda i: (0, 0, 0)),
            pl.BlockSpec((NC, 1, CW), lambda i: (0, 0, 0)),
        ],
        out_specs=pl.BlockSpec((BQ, K), lambda i: (i, 0)),
        out_shape=jax.ShapeDtypeStruct((NPAD, K), jnp.int32),
        compiler_params=pltpu.CompilerParams(
            dimension_semantics=("arbitrary",)),
    )(qx, qy, qz, px, py, pz)


def _feat_body(x_ref, wc_ref, w2_ref, b_ref, ax_ref, bm_ref):
    x = x_ref[...]
    a = jnp.dot(x, wc_ref[...],
                preferred_element_type=jnp.float32) + b_ref[...]
    ax_ref[...] = jnp.concatenate([a, x], axis=1)
    bm_ref[:, 0:C] = jnp.dot(x, w2_ref[...],
                             preferred_element_type=jnp.float32)


def _feat(x, wc, w2, b):
    br = 1024
    return pl.pallas_call(
        _feat_body,
        grid=(NPAD // br,),
        in_specs=[
            pl.BlockSpec((br, C), lambda i: (i, 0)),
            pl.BlockSpec((C, C), lambda i: (0, 0)),
            pl.BlockSpec((C, C), lambda i: (0, 0)),
            pl.BlockSpec((1, C), lambda i: (0, 0)),
        ],
        out_specs=[
            pl.BlockSpec((br, 2 * C), lambda i: (i, 0)),
            pl.BlockSpec((br, 2 * C), lambda i: (i, 0)),
        ],
        out_shape=[
            jax.ShapeDtypeStruct((NPAD, 2 * C), jnp.float32),
            jax.ShapeDtypeStruct((NPAD, 2 * C), jnp.float32),
        ],
        compiler_params=pltpu.CompilerParams(
            dimension_semantics=("arbitrary",)),
    )(x, wc, w2, b)


def _gm_body(idx_hbm, bm_hbm, ax_hbm, out_hbm, idx_v, ax_v, o_v, g_v, sem):
    wid = lax.axis_index("s") * 2 + lax.axis_index("c")
    base = wid * RPW
    pltpu.sync_copy(idx_hbm.at[pl.ds(base * K, RPW * K)], idx_v)
    pltpu.sync_copy(ax_hbm.at[pl.ds(base, RPW)], ax_v)

    def step(t, _):
        pltpu.async_copy(
            bm_hbm.at[idx_v.at[pl.ds(t * GB * K, GB * K)]], g_v, sem).wait()
        for r in range(GB):
            row = t * GB + r
            for c in range(C // 16):
                sl = pl.ds(c * 16, 16)
                acc = g_v[r * K, sl]
                for n in range(1, K):
                    acc = jnp.maximum(acc, g_v[r * K + n, sl])
                o_v[row, sl] = ax_v[row, pl.ds(C + c * 16, 16)] + jnp.maximum(
                    ax_v[row, sl] + acc, 0.0)
        return 0

    lax.fori_loop(0, NG, step, 0)
    pltpu.sync_copy(o_v, out_hbm.at[pl.ds(base, RPW)])


@functools.lru_cache(maxsize=1)
def _build_gather_max():
    return functools.partial(
        pl.kernel,
        mesh=plsc.VectorSubcoreMesh(core_axis_name="c", subcore_axis_name="s"),
        out_type=jax.ShapeDtypeStruct((NPAD, C), jnp.float32),
        scratch_types=[
            pltpu.VMEM((RPW * K,), jnp.int32),
            pltpu.VMEM((RPW, 2 * C), jnp.float32),
            pltpu.VMEM((RPW, C), jnp.float32),
            pltpu.VMEM((GB * K, 2 * C), jnp.float32),
            pltpu.SemaphoreType.DMA,
        ],
    )(_gm_body)


def kernel(pillar_features, voxel_coords, W, b):
    n = pillar_features.shape[0]
    pos = voxel_coords[:, 1:4]
    pad = NPAD - n
    posq = jnp.concatenate(
        [pos, jnp.broadcast_to(pos[0], (pad, 3))], axis=0)
    posp = jnp.concatenate(
        [pos, jnp.full((pad, 3), 1e9, jnp.float32)], axis=0)
    qx = posq[:, 0:1]
    qy = posq[:, 1:2]
    qz = posq[:, 2:3]
    px = posp[:, 0].reshape(NC, 1, CW)
    py = posp[:, 1].reshape(NC, 1, CW)
    pz = posp[:, 2].reshape(NC, 1, CW)

    idx = _knn(qx, qy, qz, px, py, pz)

    xp = jnp.concatenate(
        [pillar_features, jnp.zeros((pad, C), jnp.float32)], axis=0)
    wc = W[:C] - W[C:]
    w2 = W[C:]
    ax, bm = _feat(xp, wc, w2, b.reshape(1, C))

    out = _build_gather_max()(idx.reshape(-1), bm, ax)
    return out[:n]


# speedup vs baseline: 1.1122x; 1.1122x over previous
import functools

import jax
import jax.numpy as jnp
from jax import lax
from jax.experimental import pallas as pl
from jax.experimental.pallas import tpu as pltpu
from jax.experimental.pallas import tpu_sc as plsc

N = 10000
K = 16
C = 64
NPAD = 10240
BQ = 512
CW = 256
NC = NPAD // CW
BIGI = NPAD

NW = 32
RPW = NPAD // NW
GB = 8
NG = RPW // GB


NEX = 5
FBIG = float(BIGI)
FBIG2 = float(1 << 24)


def _knn_body(qx_ref, qy_ref, qz_ref, px_ref, py_ref, pz_ref, idx_ref):
    qx = qx_ref[...]
    qy = qy_ref[...]
    qz = qz_ref[...]
    lane = lax.broadcasted_iota(
        jnp.int32, (BQ, CW), 1).astype(jnp.float32)
    lane16 = lax.broadcasted_iota(
        jnp.int32, (BQ, K), 1).astype(jnp.float32)
    lanec = lax.broadcasted_iota(
        jnp.int32, (BQ, NC), 1).astype(jnp.float32)
    inf = jnp.inf

    def dist_chunk(jj):
        dx = qx - px_ref[jj]
        dy = qy - py_ref[jj]
        dz = qz - pz_ref[jj]
        return (dx * dx + dy * dy) + dz * dz

    def extract(j, cc):
        vs, is_, vl, plast = cc
        for u in range(4):
            jj = 4 * j + u
            jf = jj.astype(jnp.float32)
            blk = dist_chunk(jj)
            excl = jnp.zeros((BQ, CW), jnp.bool_)
            vs2, is2 = [], []
            for _ in range(NEX):
                v = jnp.min(jnp.where(excl, inf, blk),
                            axis=1, keepdims=True)
                p = jnp.min(jnp.where((blk == v) & ~excl, lane, FBIG),
                            axis=1, keepdims=True)
                excl = excl | (lane == p)
                vs2.append(v)
                is2.append(p + jf * CW)
            sel = lanec == jf
            vs = [jnp.where(sel, v2, a) for a, v2 in zip(vs, vs2)]
            is_ = [jnp.where(sel, i2, a) for a, i2 in zip(is_, is2)]
            vl = jnp.where(sel, v, vl)
            plast = jnp.where(sel, p, plast)
        return vs, is_, vl, plast

    vs, is_, vl, plast = lax.fori_loop(
        0, NC // 4, extract,
        ([jnp.full((BQ, NC), inf, jnp.float32) for _ in range(NEX)],
         [jnp.full((BQ, NC), FBIG2, jnp.float32) for _ in range(NEX)],
         jnp.full((BQ, NC), inf, jnp.float32),
         jnp.full((BQ, NC), FBIG, jnp.float32)))

    def top16(comb_v, comb_i, tv, ti):
        cv = jnp.full((BQ, 1), -inf, jnp.float32)
        ci = jnp.full((BQ, 1), -1.0, jnp.float32)
        for k in range(K):
            candm = (comb_v > cv) | ((comb_v == cv) & (comb_i > ci))
            m = jnp.min(jnp.where(candm, comb_v, inf),
                        axis=1, keepdims=True)
            ai = jnp.min(jnp.where((comb_v == m) & candm, comb_i, FBIG2),
                         axis=1, keepdims=True)
            tv = jnp.where(lane16 == k, m, tv)
            ti = jnp.where(lane16 == k, ai, ti)
            cv, ci = m, ai
        return tv, ti, cv

    tv, ti, t16 = top16(
        jnp.concatenate(vs, axis=1), jnp.concatenate(is_, axis=1),
        jnp.full((BQ, K), inf, jnp.float32),
        jnp.full((BQ, K), FBIG2, jnp.float32))

    def sweep_cond(carry):
        return carry[0] > 0.5

    def sweep_body(carry):
        _, tv, ti, t16, vl, plast = carry

        def chunk_body(j, cc):
            cvs, cis, vl, plast = cc
            jf = j.astype(jnp.float32)
            blk = dist_chunk(j)
            sel = lanec == jf
            vj = jnp.max(jnp.where(sel, vl, -inf), axis=1, keepdims=True)
            pj = jnp.max(jnp.where(sel, plast, -inf), axis=1, keepdims=True)
            live = vj <= t16
            cand = ((blk > vj) | ((blk == vj) & (lane > pj))) & live
            v = jnp.min(jnp.where(cand, blk, inf), axis=1, keepdims=True)
            p = jnp.min(jnp.where((blk == v) & cand, lane, FBIG),
                        axis=1, keepdims=True)
            gid = jnp.where(v < inf, p + jf * CW, FBIG2)
            cvs = jnp.where(sel, v, cvs)
            cis = jnp.where(sel, gid, cis)
            vl = jnp.where(sel, jnp.where(live, v, vj), vl)
            plast = jnp.where(sel, jnp.where(live, p, pj), plast)
            return cvs, cis, vl, plast

        cvs, cis, vl, plast = lax.fori_loop(
            0, NC, chunk_body,
            (jnp.full((BQ, NC), inf, jnp.float32),
             jnp.full((BQ, NC), FBIG2, jnp.float32), vl, plast))
        tv, ti, t16 = top16(jnp.concatenate([tv, cvs], axis=1),
                            jnp.concatenate([ti, cis], axis=1), tv, ti)
        rem = jnp.sum((vl <= t16).astype(jnp.float32))
        return rem, tv, ti, t16, vl, plast

    rem0 = jnp.sum((vl <= t16).astype(jnp.float32))
    _, _, ti, _, _, _ = lax.while_loop(
        sweep_cond, sweep_body, (rem0, tv, ti, t16, vl, plast))
    idx_ref[...] = ti.astype(jnp.int32)


def _knn(qx, qy, qz, px, py, pz):
    return pl.pallas_call(
        _knn_body,
        grid=(NPAD // BQ,),
        in_specs=[
            pl.BlockSpec((BQ, 1), lambda i: (i, 0)),
            pl.BlockSpec((BQ, 1), lambda i: (i, 0)),
            pl.BlockSpec((BQ, 1), lambda i: (i, 0)),
            pl.BlockSpec((NC, 1, CW), lambda i: (0, 0, 0)),
            pl.BlockSpec((NC, 1, CW), lambda i: (0, 0, 0)),
            pl.BlockSpec((NC, 1, CW), lambda i: (0, 0, 0)),
        ],
        out_specs=pl.BlockSpec((BQ, K), lambda i: (i, 0)),
        out_shape=jax.ShapeDtypeStruct((NPAD, K), jnp.int32),
        compiler_params=pltpu.CompilerParams(
            dimension_semantics=("arbitrary",)),
    )(qx, qy, qz, px, py, pz)


def _feat_body(x_ref, wc_ref, w2_ref, b_ref, ax_ref, bm_ref):
    x = x_ref[...]
    a = jnp.dot(x, wc_ref[...],
                preferred_element_type=jnp.float32) + b_ref[...]
    ax_ref[...] = jnp.concatenate([a, x], axis=1)
    bm_ref[:, 0:C] = jnp.dot(x, w2_ref[...],
                             preferred_element_type=jnp.float32)


def _feat(x, wc, w2, b):
    br = 1024
    return pl.pallas_call(
        _feat_body,
        grid=(NPAD // br,),
        in_specs=[
            pl.BlockSpec((br, C), lambda i: (i, 0)),
            pl.BlockSpec((C, C), lambda i: (0, 0)),
            pl.BlockSpec((C, C), lambda i: (0, 0)),
            pl.BlockSpec((1, C), lambda i: (0, 0)),
        ],
        out_specs=[
            pl.BlockSpec((br, 2 * C), lambda i: (i, 0)),
            pl.BlockSpec((br, 2 * C), lambda i: (i, 0)),
        ],
        out_shape=[
            jax.ShapeDtypeStruct((NPAD, 2 * C), jnp.float32),
            jax.ShapeDtypeStruct((NPAD, 2 * C), jnp.float32),
        ],
        compiler_params=pltpu.CompilerParams(
            dimension_semantics=("arbitrary",)),
    )(x, wc, w2, b)


def _gm_body(idx_hbm, bm_hbm, ax_hbm, out_hbm, idx_v, ax_v, o_v, g_v, sem):
    wid = lax.axis_index("s") * 2 + lax.axis_index("c")
    base = wid * RPW
    pltpu.sync_copy(idx_hbm.at[pl.ds(base * K, RPW * K)], idx_v)
    pltpu.sync_copy(ax_hbm.at[pl.ds(base, RPW)], ax_v)

    def step(t, _):
        pltpu.async_copy(
            bm_hbm.at[idx_v.at[pl.ds(t * GB * K, GB * K)]], g_v, sem).wait()
        for r in range(GB):
            row = t * GB + r
            for c in range(C // 16):
                sl = pl.ds(c * 16, 16)
                acc = g_v[r * K, sl]
                for n in range(1, K):
                    acc = jnp.maximum(acc, g_v[r * K + n, sl])
                o_v[row, sl] = ax_v[row, pl.ds(C + c * 16, 16)] + jnp.maximum(
                    ax_v[row, sl] + acc, 0.0)
        return 0

    lax.fori_loop(0, NG, step, 0)
    pltpu.sync_copy(o_v, out_hbm.at[pl.ds(base, RPW)])


@functools.lru_cache(maxsize=1)
def _build_gather_max():
    return functools.partial(
        pl.kernel,
        mesh=plsc.VectorSubcoreMesh(core_axis_name="c", subcore_axis_name="s"),
        out_type=jax.ShapeDtypeStruct((NPAD, C), jnp.float32),
        scratch_types=[
            pltpu.VMEM((RPW * K,), jnp.int32),
            pltpu.VMEM((RPW, 2 * C), jnp.float32),
            pltpu.VMEM((RPW, C), jnp.float32),
            pltpu.VMEM((GB * K, 2 * C), jnp.float32),
            pltpu.SemaphoreType.DMA,
        ],
    )(_gm_body)


def kernel(pillar_features, voxel_coords, W, b):
    n = pillar_features.shape[0]
    pos = voxel_coords[:, 1:4]
    pad = NPAD - n
    posq = jnp.concatenate(
        [pos, jnp.broadcast_to(pos[0], (pad, 3))], axis=0)
    posp = jnp.concatenate(
        [pos, jnp.full((pad, 3), 1e9, jnp.float32)], axis=0)
    qx = posq[:, 0:1]
    qy = posq[:, 1:2]
    qz = posq[:, 2:3]
    px = posp[:, 0].reshape(NC, 1, CW)
    py = posp[:, 1].reshape(NC, 1, CW)
    pz = posp[:, 2].reshape(NC, 1, CW)

    idx = _knn(qx, qy, qz, px, py, pz)

    xp = jnp.concatenate(
        [pillar_features, jnp.zeros((pad, C), jnp.float32)], axis=0)
    wc = W[:C] - W[C:]
    w2 = W[C:]
    ax, bm = _feat(xp, wc, w2, b.reshape(1, C))

    out = _build_gather_max()(idx.reshape(-1), bm, ax)
    return out[:n]
